# baseline (speedup 1.0000x reference)
import functools

import jax
import jax.numpy as jnp
from jax import lax
from jax.experimental import pallas as pl
from jax.experimental.pallas import tpu as pltpu
from jax.experimental.pallas import tpu_sc as plsc

_MAX_LENGTH = 200


def _sc_broadcast(table, batch, seq, nrows, dim):
    vp, lanes = table.shape
    seq_pad = (seq + 15) // 16 * 16
    sd = seq * dim
    mesh = plsc.VectorSubcoreMesh(core_axis_name="c", subcore_axis_name="s")
    nw = 32
    rows_per_w = batch // nw
    tile_rows = 4
    ndma = rows_per_w // tile_rows

    @functools.partial(
        pl.kernel, mesh=mesh,
        out_type=jax.ShapeDtypeStruct((batch, sd), jnp.float32),
        scratch_types=[
            pltpu.VMEM((seq_pad,), jnp.int32),
            pltpu.VMEM((seq_pad, lanes), jnp.float32),
            pltpu.VMEM((tile_rows, sd), jnp.float32),
            pltpu.SemaphoreType.DMA,
            pltpu.SemaphoreType.DMA,
        ],
    )
    def k(table_hbm, out_hbm, idx_v, rows_v, tile_v, gsem, wsem):
        wid = lax.axis_index("s") * 2 + lax.axis_index("c")

        for i in range(seq_pad // 16):
            base = lax.iota(jnp.int32, 16) + (i * 16)
            pos = jnp.minimum(jnp.maximum(base, _MAX_LENGTH), nrows - 1)
            idx_v[pl.ds(i * 16, 16)] = pos

        pltpu.async_copy(table_hbm.at[idx_v], rows_v, gsem).wait()

        def _pack(s, carry):
            for c in range(dim // 16):
                v = rows_v[s, pl.ds(c * 16, 16)]
                for j in range(tile_rows):
                    tile_v[j, pl.ds(s * dim + c * 16, 16)] = v
            return carry

        lax.fori_loop(0, seq, _pack, 0)

        base_row = wid * rows_per_w
        copies = [
            pltpu.make_async_copy(
                tile_v,
                out_hbm.at[pl.ds(base_row + j * tile_rows, tile_rows)],
                wsem)
            for j in range(ndma)
        ]
        for cp in copies:
            cp.start()
        for cp in copies:
            cp.wait()

    return k(table)


def kernel(inputs, kernel):
    batch, seq = inputs.shape
    nrows, dim = kernel.shape
    vp = (nrows + 7) // 8 * 8
    lanes = max(dim, 128)
    table = jnp.zeros((vp, lanes), kernel.dtype).at[:nrows, :dim].set(kernel)

    out = _sc_broadcast(table, batch, seq, nrows, dim)
    return out.reshape(batch, seq, dim)
